# Initial kernel scaffold; baseline (speedup 1.0000x reference)
import jax
import jax.numpy as jnp
from jax.experimental import pallas as pl


def kernel(x, conv_w, conv_b, w1, b1, w2, b2, wfc, bfc):
    raise NotImplementedError("write your pallas kernel here")


# speedup vs baseline: 2.0200x; 2.0200x over previous
import numpy as np
import jax
import jax.numpy as jnp
from jax.experimental import pallas as pl
from jax.experimental.pallas import tpu as pltpu

B = 8
IN_FEATS = 256
HID = 512
NUM_CLASSES = 1000
NUM_NODES = 16
NODE_DIM = IN_FEATS // NUM_NODES
H = W = 224
KPAD = 32


def _conv_mean_body(x_ref, w_ref, o_ref):
    ones_rows = jnp.ones((5, W), dtype=jnp.bfloat16)

    def block_step(i, acc):
        y0 = pl.multiple_of(i * 8, 8)
        win = [x_ref[0, ci, pl.ds(y0, 16), :] for ci in range(3)]
        for r in range(8):
            pieces = []
            for ci in range(3):
                for dx in range(3):
                    pieces.append(win[ci][r:r + 3, dx:dx + W])
            pieces.append(ones_rows)
            pt = jnp.concatenate(pieces, axis=0)
            z = jax.lax.dot_general(
                pt, w_ref[...],
                dimension_numbers=(((0,), (0,)), ((), ())),
                preferred_element_type=jnp.float32)
            acc = acc + jnp.sum(jnp.maximum(z, 0.0), axis=0)
        return acc

    acc = jax.lax.fori_loop(0, H // 8, block_step,
                            jnp.zeros((IN_FEATS,), jnp.float32))
    o_ref[0, 0, :] = acc * jnp.float32(1.0 / (H * W))


def _gcn_tail_body(nodes_ref, a_ref, p_ref, w1t_ref, b1_ref, w2t_ref, b2_ref,
                   wfct_ref, bfc_ref, o_ref):
    f32 = jnp.float32
    nodes = nodes_ref[...]
    agg1 = jax.lax.dot_general(
        a_ref[...], nodes, (((1,), (0,)), ((), ())), preferred_element_type=f32)
    h1 = jnp.maximum(
        jax.lax.dot_general(agg1, w1t_ref[...], (((1,), (0,)), ((), ())),
                            preferred_element_type=f32) + b1_ref[...], 0.0)
    agg2 = jax.lax.dot_general(
        a_ref[...], h1, (((1,), (0,)), ((), ())), preferred_element_type=f32)
    h2 = jnp.maximum(
        jax.lax.dot_general(agg2, w2t_ref[...], (((1,), (0,)), ((), ())),
                            preferred_element_type=f32) + b2_ref[...], 0.0)
    pooled = jax.lax.dot_general(
        p_ref[...], h2, (((1,), (0,)), ((), ())), preferred_element_type=f32)
    o_ref[...] = jax.lax.dot_general(
        pooled, wfct_ref[...], (((1,), (0,)), ((), ())),
        preferred_element_type=f32) + bfc_ref[...]


def _agg_matrix():
    a16 = np.zeros((NUM_NODES, NUM_NODES), np.float32)
    for i in range(NUM_NODES - 1):
        a16[i, i + 1:] = 1.0 / (NUM_NODES - 1 - i)
    a = np.kron(np.eye(B, dtype=np.float32), a16)
    return jnp.asarray(a)


def _pool_matrix():
    p = np.kron(np.eye(B, dtype=np.float32),
                np.full((1, NUM_NODES), 1.0 / NUM_NODES, np.float32))
    return jnp.asarray(p)


def kernel(x, conv_w, conv_b, w1, b1, w2, b2, wfc, bfc):
    xp = jnp.pad(x, ((0, 0), (0, 0), (1, 15), (1, 31))).astype(jnp.bfloat16)
    wmat = conv_w.transpose(1, 3, 2, 0).reshape(27, IN_FEATS)
    wmat = jnp.concatenate(
        [wmat, conv_b[None, :], jnp.zeros((4, IN_FEATS), conv_b.dtype)], axis=0)
    wmat = wmat.astype(jnp.bfloat16)

    h = pl.pallas_call(
        _conv_mean_body,
        grid=(B,),
        in_specs=[
            pl.BlockSpec((1, 3, 240, 256), lambda i: (i, 0, 0, 0)),
            pl.BlockSpec((KPAD, IN_FEATS), lambda i: (0, 0)),
        ],
        out_specs=pl.BlockSpec((1, 1, IN_FEATS), lambda i: (i, 0, 0)),
        out_shape=jax.ShapeDtypeStruct((B, 1, IN_FEATS), jnp.float32),
    )(xp, wmat)

    nodes = h.reshape(B * NUM_NODES, NODE_DIM)

    out = pl.pallas_call(
        _gcn_tail_body,
        out_shape=jax.ShapeDtypeStruct((B, NUM_CLASSES), jnp.float32),
    )(nodes, _agg_matrix(), _pool_matrix(),
      w1.T, b1[None, :], w2.T, b2[None, :], wfc.T, bfc[None, :])
    return out


# speedup vs baseline: 2.0707x; 1.0251x over previous
import numpy as np
import jax
import jax.numpy as jnp
from jax.experimental import pallas as pl
from jax.experimental.pallas import tpu as pltpu

B = 8
IN_FEATS = 256
HID = 512
NUM_CLASSES = 1000
NUM_NODES = 16
NODE_DIM = IN_FEATS // NUM_NODES
H = W = 224
KPAD = 32


def _conv_mean_body(x_ref, w_ref, o_ref):
    ones_rows = jnp.ones((5, W), dtype=jnp.bfloat16)

    def block_step(i, acc):
        y0 = pl.multiple_of(i * 8, 8)
        win = [x_ref[0, ci, pl.ds(y0, 16), :] for ci in range(3)]
        for r in range(8):
            pieces = []
            for ci in range(3):
                for dx in range(3):
                    pieces.append(win[ci][r:r + 3, dx:dx + W])
            pieces.append(ones_rows)
            pt = jnp.concatenate(pieces, axis=0)
            z = jax.lax.dot_general(
                pt, w_ref[...],
                dimension_numbers=(((0,), (0,)), ((), ())),
                preferred_element_type=jnp.float32)
            acc = acc + jnp.sum(
                jnp.maximum(z, 0.0).reshape(W // 8, 8, IN_FEATS), axis=0)
        return acc

    acc = jax.lax.fori_loop(0, H // 8, block_step,
                            jnp.zeros((8, IN_FEATS), jnp.float32))
    o_ref[0, 0, :] = jnp.sum(acc, axis=0) * jnp.float32(1.0 / (H * W))


def _gcn_tail_body(nodes_ref, a_ref, p_ref, w1t_ref, b1_ref, w2t_ref, b2_ref,
                   wfct_ref, bfc_ref, o_ref):
    f32 = jnp.float32
    nodes = nodes_ref[...]
    agg1 = jax.lax.dot_general(
        a_ref[...], nodes, (((1,), (0,)), ((), ())), preferred_element_type=f32)
    h1 = jnp.maximum(
        jax.lax.dot_general(agg1, w1t_ref[...], (((1,), (0,)), ((), ())),
                            preferred_element_type=f32) + b1_ref[...], 0.0)
    agg2 = jax.lax.dot_general(
        a_ref[...], h1, (((1,), (0,)), ((), ())), preferred_element_type=f32)
    h2 = jnp.maximum(
        jax.lax.dot_general(agg2, w2t_ref[...], (((1,), (0,)), ((), ())),
                            preferred_element_type=f32) + b2_ref[...], 0.0)
    pooled = jax.lax.dot_general(
        p_ref[...], h2, (((1,), (0,)), ((), ())), preferred_element_type=f32)
    o_ref[...] = jax.lax.dot_general(
        pooled, wfct_ref[...], (((1,), (0,)), ((), ())),
        preferred_element_type=f32) + bfc_ref[...]


def _agg_matrix():
    a16 = np.zeros((NUM_NODES, NUM_NODES), np.float32)
    for i in range(NUM_NODES - 1):
        a16[i, i + 1:] = 1.0 / (NUM_NODES - 1 - i)
    a = np.kron(np.eye(B, dtype=np.float32), a16)
    return jnp.asarray(a)


def _pool_matrix():
    p = np.kron(np.eye(B, dtype=np.float32),
                np.full((1, NUM_NODES), 1.0 / NUM_NODES, np.float32))
    return jnp.asarray(p)


def kernel(x, conv_w, conv_b, w1, b1, w2, b2, wfc, bfc):
    xp = jnp.pad(x, ((0, 0), (0, 0), (1, 15), (1, 31))).astype(jnp.bfloat16)
    wmat = conv_w.transpose(1, 3, 2, 0).reshape(27, IN_FEATS)
    wmat = jnp.concatenate(
        [wmat, conv_b[None, :], jnp.zeros((4, IN_FEATS), conv_b.dtype)], axis=0)
    wmat = wmat.astype(jnp.bfloat16)

    h = pl.pallas_call(
        _conv_mean_body,
        grid=(B,),
        in_specs=[
            pl.BlockSpec((1, 3, 240, 256), lambda i: (i, 0, 0, 0)),
            pl.BlockSpec((KPAD, IN_FEATS), lambda i: (0, 0)),
        ],
        out_specs=pl.BlockSpec((1, 1, IN_FEATS), lambda i: (i, 0, 0)),
        out_shape=jax.ShapeDtypeStruct((B, 1, IN_FEATS), jnp.float32),
        compiler_params=pltpu.CompilerParams(
            dimension_semantics=("parallel",)),
    )(xp, wmat)

    nodes = h.reshape(B * NUM_NODES, NODE_DIM)

    out = pl.pallas_call(
        _gcn_tail_body,
        out_shape=jax.ShapeDtypeStruct((B, NUM_CLASSES), jnp.float32),
    )(nodes, _agg_matrix(), _pool_matrix(),
      w1.T, b1[None, :], w2.T, b2[None, :], wfc.T, bfc[None, :])
    return out
